# Initial kernel scaffold; baseline (speedup 1.0000x reference)
import jax
import jax.numpy as jnp
from jax.experimental import pallas as pl


def kernel(x, edge_index, W_ft, b_ft, W_conv, b_conv, velocity_params, attention, bn_gamma, bn_beta, W1, b1, ln_g, ln_b, W2, b2):
    raise NotImplementedError("write your pallas kernel here")


# speedup vs baseline: 3.0224x; 3.0224x over previous
import functools

import jax
import jax.numpy as jnp
from jax import lax
from jax.experimental import pallas as pl
from jax.experimental.pallas import tpu as pltpu
from jax.experimental.pallas import tpu_sc as plsc

N = 10000
E = 320000
FEAT = 128
HALF = 64
NOBS = 4
VMAX = 0.9

E_PAD = 327680
GEOM_PER_TILE = E_PAD // 32
GEOM_ITERS = GEOM_PER_TILE // 16

SUB = 128
EROWS = E_PAD // SUB
ROWS_PER_TILE = EROWS // 16
ACC_ROWS = 10240
TRASH_ROW = 10000
ZERO_ROWS = ACC_ROWS // 16


def _newton_rsqrt(d2):
    magic = jnp.full((16,), 0x5F3759DF, jnp.int32)
    yi = magic - lax.shift_right_arithmetic(plsc.bitcast(d2, jnp.int32),
                                            jnp.full((16,), 1, jnp.int32))
    y = plsc.bitcast(yi, jnp.float32)
    for _ in range(3):
        y = y * (1.5 - 0.5 * d2 * y * y)
    return y


def _geom_body(px, py, pz, src_h, dst_h, dist_h, dsum_h, tmax_h,
               px_b, py_b, pz_b, src_b, dst_b, dist_b, dsum_b, max_b):
    c = lax.axis_index("c")
    s = lax.axis_index("s")
    wid = c * 16 + s
    e0 = pl.multiple_of(wid * GEOM_PER_TILE, GEOM_PER_TILE)
    pltpu.sync_copy(px, px_b)
    pltpu.sync_copy(py, py_b)
    pltpu.sync_copy(pz, pz_b)
    pltpu.sync_copy(src_h.at[pl.ds(e0, GEOM_PER_TILE)], src_b)
    pltpu.sync_copy(dst_h.at[pl.ds(e0, GEOM_PER_TILE)], dst_b)

    def body(i, vmax):
        sl = pl.ds(i * 16, 16)
        si = src_b[sl]
        di = dst_b[sl]
        dx = plsc.load_gather(px_b, [si]) - plsc.load_gather(px_b, [di])
        dy = plsc.load_gather(py_b, [si]) - plsc.load_gather(py_b, [di])
        dz = plsc.load_gather(pz_b, [si]) - plsc.load_gather(pz_b, [di])
        d2 = dx * dx + dy * dy + dz * dz + 1e-20
        dist = d2 * _newton_rsqrt(d2)
        dist_b[sl] = dist
        dsum_b[sl] = (dx + dy + dz) / (dist + 1e-8)
        return jnp.maximum(vmax, dist)

    vmax = lax.fori_loop(0, GEOM_ITERS, body, jnp.zeros((16,), jnp.float32))
    for r in range(8):
        for q in range(8):
            max_b[r, pl.ds(q * 16, 16)] = vmax
    pltpu.sync_copy(dist_b, dist_h.at[pl.ds(e0, GEOM_PER_TILE)])
    pltpu.sync_copy(dsum_b, dsum_h.at[pl.ds(e0, GEOM_PER_TILE)])
    pltpu.sync_copy(max_b, tmax_h.at[pl.ds(wid * 8, 8)])


def _scatter_body(t_h, srcr_h, dstr_h, w_h, zeros_h, out_h,
                  acc, src_g, dst_g, w_g, t_b, sem):
    c = lax.axis_index("c")
    s = lax.axis_index("s")
    base = s * ROWS_PER_TILE

    for p in range(2):
        o = 2 * c + p
        z0 = s * ZERO_ROWS
        pltpu.sync_copy(zeros_h.at[pl.ds(z0, ZERO_ROWS)],
                        acc.at[pl.ds(z0, ZERO_ROWS)])
        plsc.subcore_barrier()

        def gloop(g, _):
            r0 = pl.multiple_of(base + g * 8, 8)
            pltpu.sync_copy(srcr_h.at[pl.ds(r0, 8)], src_g)
            pltpu.sync_copy(dstr_h.at[pl.ds(r0, 8)], dst_g)
            pltpu.sync_copy(w_h.at[o, pl.ds(r0, 8)], w_g)

            def jloop(j, _):
                pltpu.async_copy(t_h.at[src_g.at[j]], t_b, sem).wait()
                for gg in range(SUB // 16):
                    wv = w_g[j, pl.ds(gg * 16, 16)]
                    for k in range(16):
                        wk = wv[k]
                        e = gg * 16 + k
                        for q in range(FEAT // 16):
                            fs = pl.ds(q * 16, 16)
                            t_b[e, fs] = t_b[e, fs] * wk
                pltpu.sync_copy(t_b, acc.at[dst_g.at[j]], add=True)
                return 0
            lax.fori_loop(0, 8, jloop, 0)
            return 0
        lax.fori_loop(0, ROWS_PER_TILE // 8, gloop, 0)
        plsc.subcore_barrier()

        col = pl.multiple_of(o * FEAT, FEAT)
        f0 = s * ZERO_ROWS
        pltpu.sync_copy(acc.at[pl.ds(f0, 400)],
                        out_h.at[pl.ds(f0, 400), pl.ds(col, FEAT)])

        @pl.when(s < 15)
        def _rest():
            pltpu.sync_copy(acc.at[pl.ds(f0 + 400, 240)],
                            out_h.at[pl.ds(f0 + 400, 240), pl.ds(col, FEAT)])
        plsc.subcore_barrier()


def _pre_body(x_ref, wft_ref, bft_ref, wcv_ref, bcv_ref, t_ref):
    h = jnp.dot(x_ref[...], wft_ref[...],
                preferred_element_type=jnp.float32) + bft_ref[...]
    t_ref[...] = jnp.dot(h, wcv_ref[...],
                         preferred_element_type=jnp.float32) + bcv_ref[...]


def _weights_body(dist_ref, dsum_ref, tmax_ref, vp_ref, w_ref):
    maxd = jnp.max(tmax_ref[...])
    dn = jnp.clip(dist_ref[...] / maxd, 0.0, VMAX)
    ds = dsum_ref[...]
    rows = []
    for o in range(NOBS):
        vr = vp_ref[o] * dn
        a = jnp.sqrt(jnp.clip(1.0 - vr * vr, 1e-12, None)) / (1.0 + vr * ds)
        rows.append(a[None])
    w_ref[...] = jnp.concatenate(rows, axis=0)


def _post_body(comb_ref, gam_ref, bet_ref, att_ref, w1t_ref, b1_ref,
               lng_ref, lnb_ref, w2t_ref, b2_ref, out_ref, ssum, ssq):
    p = pl.program_id(0)
    i = pl.program_id(1)

    @pl.when(p == 0)
    def _stats():
        xb = comb_ref[...]
        sv = jnp.sum(xb, axis=0, keepdims=True)
        qv = jnp.sum(xb * xb, axis=0, keepdims=True)

        @pl.when(i == 0)
        def _init():
            ssum[...] = sv
            ssq[...] = qv

        @pl.when(i > 0)
        def _acc():
            ssum[...] += sv
            ssq[...] += qv

    @pl.when(p == 1)
    def _compute():
        xb = comb_ref[...]
        inv_n = 1.0 / N
        mean = ssum[...] * inv_n
        var = ssq[...] * inv_n - mean * mean
        istd = lax.rsqrt(var + 1e-5)
        a = gam_ref[...] * istd
        b = bet_ref[...] - mean * a
        z0 = jnp.maximum(xb * a + b, 0.0) * att_ref[...]
        z1 = jnp.dot(z0, w1t_ref[...],
                     preferred_element_type=jnp.float32) + b1_ref[...]
        mu = jnp.mean(z1, axis=1, keepdims=True)
        zc = z1 - mu
        v = jnp.mean(zc * zc, axis=1, keepdims=True)
        z = zc * lax.rsqrt(v + 1e-5) * lng_ref[...] + lnb_ref[...]
        z = jnp.maximum(z, 0.0)
        out_ref[...] = jnp.dot(z, w2t_ref[...],
                               preferred_element_type=jnp.float32) + b2_ref[...]

    @pl.when(p == 0)
    def _junk():
        out_ref[...] = jnp.zeros_like(out_ref)


def kernel(x, edge_index, W_ft, b_ft, W_conv, b_conv, velocity_params,
           attention, bn_gamma, bn_beta, W1, b1, ln_g, ln_b, W2, b2):
    f32 = jnp.float32
    mesh = plsc.VectorSubcoreMesh(core_axis_name="c", subcore_axis_name="s")

    t = pl.pallas_call(
        _pre_body,
        grid=(10,),
        in_specs=[
            pl.BlockSpec((1000, FEAT), lambda i: (i, 0)),
            pl.BlockSpec((FEAT, FEAT), lambda i: (0, 0)),
            pl.BlockSpec((1, FEAT), lambda i: (0, 0)),
            pl.BlockSpec((FEAT, FEAT), lambda i: (0, 0)),
            pl.BlockSpec((1, FEAT), lambda i: (0, 0)),
        ],
        out_specs=pl.BlockSpec((1000, FEAT), lambda i: (i, 0)),
        out_shape=jax.ShapeDtypeStruct((N, FEAT), f32),
    )(x, W_ft.T, b_ft.reshape(1, FEAT), W_conv.T, b_conv.reshape(1, FEAT))

    src = edge_index[0]
    dst = edge_index[1]
    pad = E_PAD - E
    src_p = jnp.concatenate([src, jnp.zeros((pad,), jnp.int32)])
    dst_p = jnp.concatenate([dst, jnp.full((pad,), TRASH_ROW, jnp.int32)])

    geom = pl.kernel(
        _geom_body,
        out_type=[
            jax.ShapeDtypeStruct((E_PAD,), f32),
            jax.ShapeDtypeStruct((E_PAD,), f32),
            jax.ShapeDtypeStruct((256, 128), f32),
        ],
        mesh=mesh,
        scratch_types=[
            pltpu.VMEM((N,), f32),
            pltpu.VMEM((N,), f32),
            pltpu.VMEM((N,), f32),
            pltpu.VMEM((GEOM_PER_TILE,), jnp.int32),
            pltpu.VMEM((GEOM_PER_TILE,), jnp.int32),
            pltpu.VMEM((GEOM_PER_TILE,), f32),
            pltpu.VMEM((GEOM_PER_TILE,), f32),
            pltpu.VMEM((8, 128), f32),
        ],
        compiler_params=pltpu.CompilerParams(needs_layout_passes=False),
    )
    px = x[:, 0]
    py = x[:, 1]
    pz = x[:, 2]
    dist_p, dsum_p, tmax = geom(px, py, pz, src_p, dst_p)

    w4 = pl.pallas_call(
        _weights_body,
        grid=(10,),
        in_specs=[
            pl.BlockSpec((256, 128), lambda i: (i, 0)),
            pl.BlockSpec((256, 128), lambda i: (i, 0)),
            pl.BlockSpec((256, 128), lambda i: (0, 0)),
            pl.BlockSpec(memory_space=pltpu.SMEM),
        ],
        out_specs=pl.BlockSpec((NOBS, 256, 128), lambda i: (0, i, 0)),
        out_shape=jax.ShapeDtypeStruct((NOBS, E_PAD // 128, 128), f32),
    )(dist_p.reshape(E_PAD // 128, 128), dsum_p.reshape(E_PAD // 128, 128),
      tmax, velocity_params)

    srcr = src_p.reshape(EROWS, SUB)
    dstr = dst_p.reshape(EROWS, SUB)
    zeros_acc = jnp.zeros((ACC_ROWS, FEAT), f32)
    scatter = pl.kernel(
        _scatter_body,
        out_type=jax.ShapeDtypeStruct((N, NOBS * FEAT), f32),
        mesh=mesh,
        scratch_types=[
            pltpu.VMEM_SHARED((ACC_ROWS, FEAT), f32),
            pltpu.VMEM((8, SUB), jnp.int32),
            pltpu.VMEM((8, SUB), jnp.int32),
            pltpu.VMEM((8, SUB), f32),
            pltpu.VMEM((SUB, FEAT), f32),
            pltpu.SemaphoreType.DMA,
        ],
    )
    combined = scatter(t, srcr, dstr, w4, zeros_acc)

    out = pl.pallas_call(
        _post_body,
        grid=(2, 10),
        in_specs=[
            pl.BlockSpec((1000, 512), lambda p, i: (i, 0)),
            pl.BlockSpec((1, 512), lambda p, i: (0, 0)),
            pl.BlockSpec((1, 512), lambda p, i: (0, 0)),
            pl.BlockSpec((1, 512), lambda p, i: (0, 0)),
            pl.BlockSpec((512, 256), lambda p, i: (0, 0)),
            pl.BlockSpec((1, 256), lambda p, i: (0, 0)),
            pl.BlockSpec((1, 256), lambda p, i: (0, 0)),
            pl.BlockSpec((1, 256), lambda p, i: (0, 0)),
            pl.BlockSpec((256, 128), lambda p, i: (0, 0)),
            pl.BlockSpec((1, 128), lambda p, i: (0, 0)),
        ],
        out_specs=pl.BlockSpec((1000, 128), lambda p, i: (i, 0)),
        out_shape=jax.ShapeDtypeStruct((N, 128), f32),
        scratch_shapes=[pltpu.VMEM((1, 512), f32), pltpu.VMEM((1, 512), f32)],
    )(combined, bn_gamma.reshape(1, 512), bn_beta.reshape(1, 512),
      attention.reshape(1, 512), W1.T, b1.reshape(1, 256),
      ln_g.reshape(1, 256), ln_b.reshape(1, 256), W2.T, b2.reshape(1, 128))
    return out


# speedup vs baseline: 3.3590x; 1.1114x over previous
import functools

import jax
import jax.numpy as jnp
from jax import lax
from jax.experimental import pallas as pl
from jax.experimental.pallas import tpu as pltpu
from jax.experimental.pallas import tpu_sc as plsc

N = 10000
E = 320000
FEAT = 128
HALF = 64
NOBS = 4
VMAX = 0.9

E_PAD = 327680
GEOM_PER_TILE = E_PAD // 32
GEOM_ITERS = GEOM_PER_TILE // 16

SUB = 128
EROWS = E_PAD // SUB
ROWS_PER_TILE = EROWS // 16
ACC_ROWS = 10240
TRASH_ROW = 10000
ZERO_ROWS = ACC_ROWS // 16


def _newton_rsqrt(d2):
    magic = jnp.full((16,), 0x5F3759DF, jnp.int32)
    yi = magic - lax.shift_right_arithmetic(plsc.bitcast(d2, jnp.int32),
                                            jnp.full((16,), 1, jnp.int32))
    y = plsc.bitcast(yi, jnp.float32)
    for _ in range(3):
        y = y * (1.5 - 0.5 * d2 * y * y)
    return y


def _geom_body(px, py, pz, src_h, dst_h, dist_h, dsum_h, tmax_h,
               px_b, py_b, pz_b, src_b, dst_b, dist_b, dsum_b, max_b):
    c = lax.axis_index("c")
    s = lax.axis_index("s")
    wid = c * 16 + s
    e0 = pl.multiple_of(wid * GEOM_PER_TILE, GEOM_PER_TILE)
    pltpu.sync_copy(px, px_b)
    pltpu.sync_copy(py, py_b)
    pltpu.sync_copy(pz, pz_b)
    pltpu.sync_copy(src_h.at[pl.ds(e0, GEOM_PER_TILE)], src_b)
    pltpu.sync_copy(dst_h.at[pl.ds(e0, GEOM_PER_TILE)], dst_b)

    def body(i, vmax):
        sl = pl.ds(i * 16, 16)
        si = src_b[sl]
        di = dst_b[sl]
        dx = plsc.load_gather(px_b, [si]) - plsc.load_gather(px_b, [di])
        dy = plsc.load_gather(py_b, [si]) - plsc.load_gather(py_b, [di])
        dz = plsc.load_gather(pz_b, [si]) - plsc.load_gather(pz_b, [di])
        d2 = dx * dx + dy * dy + dz * dz + 1e-20
        dist = d2 * _newton_rsqrt(d2)
        dist_b[sl] = dist
        dsum_b[sl] = (dx + dy + dz) / (dist + 1e-8)
        return jnp.maximum(vmax, dist)

    vmax = lax.fori_loop(0, GEOM_ITERS, body, jnp.zeros((16,), jnp.float32))
    for r in range(8):
        for q in range(8):
            max_b[r, pl.ds(q * 16, 16)] = vmax
    pltpu.sync_copy(dist_b, dist_h.at[pl.ds(e0, GEOM_PER_TILE)])
    pltpu.sync_copy(dsum_b, dsum_h.at[pl.ds(e0, GEOM_PER_TILE)])
    pltpu.sync_copy(max_b, tmax_h.at[pl.ds(wid * 8, 8)])


def _scale_rows(t_b, slot, w_g, j):
    def gg_body(gg, _):
        wv = w_g[j, pl.ds(gg * 16, 16)]
        for k in range(16):
            wk = wv[k]
            e = gg * 16 + k
            for q in range(FEAT // 16):
                fs = pl.ds(q * 16, 16)
                t_b[slot, e, fs] = t_b[slot, e, fs] * wk
        return 0
    lax.fori_loop(0, SUB // 16, gg_body, 0)


def _scatter_body(t_h, srcr_h, dstr_h, w_h, zeros_h, out_h,
                  acc, src_g, dst_g, w_g, t_b, gsem, ssem):
    c = lax.axis_index("c")
    s = lax.axis_index("s")
    base = s * ROWS_PER_TILE

    for p in range(2):
        o = 2 * c + p
        z0 = s * ZERO_ROWS
        pltpu.sync_copy(zeros_h.at[pl.ds(z0, ZERO_ROWS)],
                        acc.at[pl.ds(z0, ZERO_ROWS)])
        plsc.subcore_barrier()

        def gloop(g, _):
            r0 = pl.multiple_of(base + g * 8, 8)
            pltpu.sync_copy(srcr_h.at[pl.ds(r0, 8)], src_g)
            pltpu.sync_copy(dstr_h.at[pl.ds(r0, 8)], dst_g)
            pltpu.sync_copy(w_h.at[o, pl.ds(r0, 8)], w_g)
            cps = {0: pltpu.async_copy(t_h.at[src_g.at[0]], t_b.at[0], gsem)}
            scps = {}
            for j in range(8):
                cps[j].wait()
                if j + 1 < 8:
                    if j - 1 >= 0:
                        scps[j - 1].wait()
                    cps[j + 1] = pltpu.async_copy(
                        t_h.at[src_g.at[j + 1]], t_b.at[(j + 1) % 2], gsem)
                _scale_rows(t_b, j % 2, w_g, j)
                scps[j] = pltpu.async_copy(
                    t_b.at[j % 2], acc.at[dst_g.at[j]], ssem, add=True)
            scps[6].wait()
            scps[7].wait()
            return 0
        lax.fori_loop(0, ROWS_PER_TILE // 8, gloop, 0)
        plsc.subcore_barrier()

        col = pl.multiple_of(o * FEAT, FEAT)
        f0 = s * ZERO_ROWS
        pltpu.sync_copy(acc.at[pl.ds(f0, 400)],
                        out_h.at[pl.ds(f0, 400), pl.ds(col, FEAT)])

        @pl.when(s < 15)
        def _rest():
            pltpu.sync_copy(acc.at[pl.ds(f0 + 400, 240)],
                            out_h.at[pl.ds(f0 + 400, 240), pl.ds(col, FEAT)])
        plsc.subcore_barrier()


def _pre_body(x_ref, wft_ref, bft_ref, wcv_ref, bcv_ref, t_ref):
    h = jnp.dot(x_ref[...], wft_ref[...],
                preferred_element_type=jnp.float32) + bft_ref[...]
    t_ref[...] = jnp.dot(h, wcv_ref[...],
                         preferred_element_type=jnp.float32) + bcv_ref[...]


def _weights_body(dist_ref, dsum_ref, tmax_ref, vp_ref, w_ref):
    maxd = jnp.max(tmax_ref[...])
    dn = jnp.clip(dist_ref[...] / maxd, 0.0, VMAX)
    ds = dsum_ref[...]
    rows = []
    for o in range(NOBS):
        vr = vp_ref[o] * dn
        a = jnp.sqrt(jnp.clip(1.0 - vr * vr, 1e-12, None)) / (1.0 + vr * ds)
        rows.append(a[None])
    w_ref[...] = jnp.concatenate(rows, axis=0)


def _post_body(comb_ref, gam_ref, bet_ref, att_ref, w1t_ref, b1_ref,
               lng_ref, lnb_ref, w2t_ref, b2_ref, out_ref, ssum, ssq):
    p = pl.program_id(0)
    i = pl.program_id(1)

    @pl.when(p == 0)
    def _stats():
        xb = comb_ref[...]
        sv = jnp.sum(xb, axis=0, keepdims=True)
        qv = jnp.sum(xb * xb, axis=0, keepdims=True)

        @pl.when(i == 0)
        def _init():
            ssum[...] = sv
            ssq[...] = qv

        @pl.when(i > 0)
        def _acc():
            ssum[...] += sv
            ssq[...] += qv

    @pl.when(p == 1)
    def _compute():
        xb = comb_ref[...]
        inv_n = 1.0 / N
        mean = ssum[...] * inv_n
        var = ssq[...] * inv_n - mean * mean
        istd = lax.rsqrt(var + 1e-5)
        a = gam_ref[...] * istd
        b = bet_ref[...] - mean * a
        z0 = jnp.maximum(xb * a + b, 0.0) * att_ref[...]
        z1 = jnp.dot(z0, w1t_ref[...],
                     preferred_element_type=jnp.float32) + b1_ref[...]
        mu = jnp.mean(z1, axis=1, keepdims=True)
        zc = z1 - mu
        v = jnp.mean(zc * zc, axis=1, keepdims=True)
        z = zc * lax.rsqrt(v + 1e-5) * lng_ref[...] + lnb_ref[...]
        z = jnp.maximum(z, 0.0)
        out_ref[...] = jnp.dot(z, w2t_ref[...],
                               preferred_element_type=jnp.float32) + b2_ref[...]

    @pl.when(p == 0)
    def _junk():
        out_ref[...] = jnp.zeros_like(out_ref)


def kernel(x, edge_index, W_ft, b_ft, W_conv, b_conv, velocity_params,
           attention, bn_gamma, bn_beta, W1, b1, ln_g, ln_b, W2, b2):
    f32 = jnp.float32
    mesh = plsc.VectorSubcoreMesh(core_axis_name="c", subcore_axis_name="s")

    t = pl.pallas_call(
        _pre_body,
        grid=(10,),
        in_specs=[
            pl.BlockSpec((1000, FEAT), lambda i: (i, 0)),
            pl.BlockSpec((FEAT, FEAT), lambda i: (0, 0)),
            pl.BlockSpec((1, FEAT), lambda i: (0, 0)),
            pl.BlockSpec((FEAT, FEAT), lambda i: (0, 0)),
            pl.BlockSpec((1, FEAT), lambda i: (0, 0)),
        ],
        out_specs=pl.BlockSpec((1000, FEAT), lambda i: (i, 0)),
        out_shape=jax.ShapeDtypeStruct((N, FEAT), f32),
    )(x, W_ft.T, b_ft.reshape(1, FEAT), W_conv.T, b_conv.reshape(1, FEAT))

    src = edge_index[0]
    dst = edge_index[1]
    pad = E_PAD - E
    src_p = jnp.concatenate([src, jnp.zeros((pad,), jnp.int32)])
    dst_p = jnp.concatenate([dst, jnp.full((pad,), TRASH_ROW, jnp.int32)])

    geom = pl.kernel(
        _geom_body,
        out_type=[
            jax.ShapeDtypeStruct((E_PAD,), f32),
            jax.ShapeDtypeStruct((E_PAD,), f32),
            jax.ShapeDtypeStruct((256, 128), f32),
        ],
        mesh=mesh,
        scratch_types=[
            pltpu.VMEM((N,), f32),
            pltpu.VMEM((N,), f32),
            pltpu.VMEM((N,), f32),
            pltpu.VMEM((GEOM_PER_TILE,), jnp.int32),
            pltpu.VMEM((GEOM_PER_TILE,), jnp.int32),
            pltpu.VMEM((GEOM_PER_TILE,), f32),
            pltpu.VMEM((GEOM_PER_TILE,), f32),
            pltpu.VMEM((8, 128), f32),
        ],
        compiler_params=pltpu.CompilerParams(needs_layout_passes=False),
    )
    px = x[:, 0]
    py = x[:, 1]
    pz = x[:, 2]
    dist_p, dsum_p, tmax = geom(px, py, pz, src_p, dst_p)

    w4 = pl.pallas_call(
        _weights_body,
        grid=(10,),
        in_specs=[
            pl.BlockSpec((256, 128), lambda i: (i, 0)),
            pl.BlockSpec((256, 128), lambda i: (i, 0)),
            pl.BlockSpec((256, 128), lambda i: (0, 0)),
            pl.BlockSpec(memory_space=pltpu.SMEM),
        ],
        out_specs=pl.BlockSpec((NOBS, 256, 128), lambda i: (0, i, 0)),
        out_shape=jax.ShapeDtypeStruct((NOBS, E_PAD // 128, 128), f32),
    )(dist_p.reshape(E_PAD // 128, 128), dsum_p.reshape(E_PAD // 128, 128),
      tmax, velocity_params)

    srcr = src_p.reshape(EROWS, SUB)
    dstr = dst_p.reshape(EROWS, SUB)
    zeros_acc = jnp.zeros((ACC_ROWS, FEAT), f32)
    scatter = pl.kernel(
        _scatter_body,
        out_type=jax.ShapeDtypeStruct((N, NOBS * FEAT), f32),
        mesh=mesh,
        scratch_types=[
            pltpu.VMEM_SHARED((ACC_ROWS, FEAT), f32),
            pltpu.VMEM((8, SUB), jnp.int32),
            pltpu.VMEM((8, SUB), jnp.int32),
            pltpu.VMEM((8, SUB), f32),
            pltpu.VMEM((2, SUB, FEAT), f32),
            pltpu.SemaphoreType.DMA,
            pltpu.SemaphoreType.DMA,
        ],
    )
    combined = scatter(t, srcr, dstr, w4, zeros_acc)

    out = pl.pallas_call(
        _post_body,
        grid=(2, 10),
        in_specs=[
            pl.BlockSpec((1000, 512), lambda p, i: (i, 0)),
            pl.BlockSpec((1, 512), lambda p, i: (0, 0)),
            pl.BlockSpec((1, 512), lambda p, i: (0, 0)),
            pl.BlockSpec((1, 512), lambda p, i: (0, 0)),
            pl.BlockSpec((512, 256), lambda p, i: (0, 0)),
            pl.BlockSpec((1, 256), lambda p, i: (0, 0)),
            pl.BlockSpec((1, 256), lambda p, i: (0, 0)),
            pl.BlockSpec((1, 256), lambda p, i: (0, 0)),
            pl.BlockSpec((256, 128), lambda p, i: (0, 0)),
            pl.BlockSpec((1, 128), lambda p, i: (0, 0)),
        ],
        out_specs=pl.BlockSpec((1000, 128), lambda p, i: (i, 0)),
        out_shape=jax.ShapeDtypeStruct((N, 128), f32),
        scratch_shapes=[pltpu.VMEM((1, 512), f32), pltpu.VMEM((1, 512), f32)],
    )(combined, bn_gamma.reshape(1, 512), bn_beta.reshape(1, 512),
      attention.reshape(1, 512), W1.T, b1.reshape(1, 256),
      ln_g.reshape(1, 256), ln_b.reshape(1, 256), W2.T, b2.reshape(1, 128))
    return out
